# baseline (speedup 1.0000x reference)
import jax
import jax.numpy as jnp
from jax.experimental import pallas as pl

NUM_PATCHES = 576
LATENT_DIM = 768
BATCH = 64

BB = 8


def _add_kernel(x_ref, pos_ref, out_ref):
    out_ref[...] = x_ref[...] + pos_ref[...]


def _half(x, pos_table, offset, nb):
    return pl.pallas_call(
        _add_kernel,
        grid=(nb // BB,),
        in_specs=[
            pl.BlockSpec((BB, NUM_PATCHES, LATENT_DIM),
                         lambda b: (b + offset // BB, 0, 0)),
            pl.BlockSpec((NUM_PATCHES, LATENT_DIM), lambda b: (0, 0)),
        ],
        out_specs=pl.BlockSpec((BB, NUM_PATCHES, LATENT_DIM), lambda b: (b, 0, 0)),
        out_shape=jax.ShapeDtypeStruct((nb, NUM_PATCHES, LATENT_DIM), x.dtype),
    )(x, pos_table)


def kernel(x, pos_table):
    lo = _half(x, pos_table, 0, 32)
    hi = _half(x, pos_table, 32, 32)
    return jnp.concatenate([lo, hi], axis=0)
